# baseline (speedup 1.0000x reference)
import functools

import jax
import jax.numpy as jnp
from jax import lax
from jax.experimental import pallas as pl
from jax.experimental.pallas import tpu as pltpu
from jax.experimental.pallas import tpu_sc as plsc

NC = 2
NS = 16
L = 16
NW = NC * NS

B = 16384
D = 64
BPW = B // NW


def _sc_body(tgt_hbm, ctx_hbm, table_hbm, out_hbm,
             tidx_v, cidx_v, trows_v, crows_v, out_v, sem_t, sem_c):
    wid = lax.axis_index("s") * NC + lax.axis_index("c")
    base = wid * BPW

    pltpu.sync_copy(tgt_hbm.at[pl.ds(base, BPW)], tidx_v)
    pltpu.sync_copy(ctx_hbm.at[pl.ds(base, BPW)], cidx_v)
    cp_t = pltpu.async_copy(table_hbm.at[tidx_v], trows_v, sem_t)
    cp_c = pltpu.async_copy(table_hbm.at[cidx_v], crows_v, sem_c)
    cp_t.wait()
    cp_c.wait()

    iota = lax.iota(jnp.int32, L)

    def blk_body(blk, carry):
        v = jnp.zeros((L,), jnp.float32)
        for j in range(L):
            r = blk * L + j
            s = jnp.zeros((L,), jnp.float32)
            for d in range(0, D, L):
                tv = trows_v[r, pl.ds(d, L)]
                cv = crows_v[r, pl.ds(d, L)]
                s = s + tv * cv
            v = jnp.where(iota == j, jnp.sum(s), v)
        out_v[pl.ds(blk * L, L)] = 1.0 / (1.0 + jnp.exp(-v))
        return carry

    lax.fori_loop(0, BPW // L, blk_body, 0)
    pltpu.sync_copy(out_v, out_hbm.at[pl.ds(base, BPW)])


_item2vec_sc = functools.partial(
    pl.kernel,
    out_type=jax.ShapeDtypeStruct((B,), jnp.float32),
    mesh=plsc.VectorSubcoreMesh(
        core_axis_name="c", subcore_axis_name="s",
        num_cores=NC, num_subcores=NS),
    scratch_types=[
        pltpu.VMEM((BPW,), jnp.int32),
        pltpu.VMEM((BPW,), jnp.int32),
        pltpu.VMEM((BPW, D), jnp.float32),
        pltpu.VMEM((BPW, D), jnp.float32),
        pltpu.VMEM((BPW,), jnp.float32),
        pltpu.SemaphoreType.DMA,
        pltpu.SemaphoreType.DMA,
    ],
    compiler_params=pltpu.CompilerParams(
        needs_layout_passes=False, use_tc_tiling_on_sc=False),
)(_sc_body)


@jax.jit
def kernel(target_i, context_j, label, shared_embedding):
    out = _item2vec_sc(target_i, context_j, shared_embedding)
    return (out, label.astype(jnp.float32))


# speedup vs baseline: 1.6199x; 1.6199x over previous
import functools

import jax
import jax.numpy as jnp
from jax import lax
from jax.experimental import pallas as pl
from jax.experimental.pallas import tpu as pltpu
from jax.experimental.pallas import tpu_sc as plsc

NC = 2
NS = 16
L = 16
NW = NC * NS

B = 16384
D = 64
BPW = B // NW
HALF = BPW // 2
K = 16


def _sc_body(tgt_hbm, ctx_hbm, table_hbm, out_hbm,
             tidx_v, cidx_v, trows_v, crows_v, out_v, sem_t, sem_c):
    wid = lax.axis_index("s") * NC + lax.axis_index("c")
    base = wid * BPW

    pltpu.sync_copy(tgt_hbm.at[pl.ds(base, BPW)], tidx_v)
    pltpu.sync_copy(ctx_hbm.at[pl.ds(base, BPW)], cidx_v)

    iota = lax.iota(jnp.int32, L)

    def half_body(h, carry):
        hbase = h * HALF

        def fire_chunk(g, c2):
            ti = tidx_v[pl.ds(hbase + g * K, K)]
            ci = cidx_v[pl.ds(hbase + g * K, K)]
            cps = []
            for j in range(K):
                r = g * K + j
                cps.append(pltpu.async_copy(
                    table_hbm.at[ti[j]], trows_v.at[r], sem_t))
                cps.append(pltpu.async_copy(
                    table_hbm.at[ci[j]], crows_v.at[r], sem_c))
            for cp in cps:
                cp.wait()
            return c2

        lax.fori_loop(0, HALF // K, fire_chunk, 0)

        def blk_body(blk, c2):
            v = jnp.zeros((L,), jnp.float32)
            for j in range(L):
                r = blk * L + j
                s = jnp.zeros((L,), jnp.float32)
                for d in range(0, D, L):
                    tv = trows_v[r, pl.ds(d, L)]
                    cv = crows_v[r, pl.ds(d, L)]
                    s = s + tv * cv
                v = jnp.where(iota == j, jnp.sum(s), v)
            out_v[pl.ds(hbase + blk * L, L)] = 1.0 / (1.0 + jnp.exp(-v))
            return c2

        lax.fori_loop(0, HALF // L, blk_body, 0)
        return carry

    lax.fori_loop(0, 2, half_body, 0)
    pltpu.sync_copy(out_v, out_hbm.at[pl.ds(base, BPW)])


_item2vec_sc = functools.partial(
    pl.kernel,
    out_type=jax.ShapeDtypeStruct((B,), jnp.float32),
    mesh=plsc.VectorSubcoreMesh(
        core_axis_name="c", subcore_axis_name="s",
        num_cores=NC, num_subcores=NS),
    scratch_types=[
        pltpu.VMEM((BPW,), jnp.int32),
        pltpu.VMEM((BPW,), jnp.int32),
        pltpu.VMEM((HALF, D), jnp.float32),
        pltpu.VMEM((HALF, D), jnp.float32),
        pltpu.VMEM((BPW,), jnp.float32),
        pltpu.SemaphoreType.DMA,
        pltpu.SemaphoreType.DMA,
    ],
    compiler_params=pltpu.CompilerParams(needs_layout_passes=False),
)(_sc_body)


@jax.jit
def kernel(target_i, context_j, label, shared_embedding):
    out = _item2vec_sc(target_i, context_j, shared_embedding)
    return (out, label.astype(jnp.float32))
